# baseline (speedup 1.0000x reference)
import functools

import jax
import jax.numpy as jnp
from jax import lax
from jax.experimental import pallas as pl
from jax.experimental.pallas import tpu as pltpu
from jax.experimental.pallas import tpu_sc as plsc

N = 2048
E = 32768
EPS = 1e-5
TOPK = (N + 1) // 2
BLK = 256
F32 = jnp.float32


def _build_adj(dst, src):
    info = plsc.get_sparse_core_info()
    nw = info.num_cores * info.num_subcores
    rows = 32
    passes = N // (nw * rows)
    ch = 16384
    nch = E // ch
    mesh = plsc.VectorSubcoreMesh(core_axis_name="c", subcore_axis_name="s")

    @functools.partial(
        pl.kernel,
        mesh=mesh,
        compiler_params=pltpu.CompilerParams(needs_layout_passes=False),
        out_type=jax.ShapeDtypeStruct((N * N,), F32),
        scratch_types=[
            pltpu.VMEM((rows * N,), F32),
            pltpu.VMEM((ch,), jnp.int32),
            pltpu.VMEM((ch,), jnp.int32),
        ],
    )
    def adj_kernel(dst_hbm, src_hbm, out_hbm, tile_v, dst_v, src_v):
        wid = lax.axis_index("s") * info.num_cores + lax.axis_index("c")
        lane = lax.iota(jnp.int32, 16)
        ones = jnp.ones((16,), F32)
        zeros16 = jnp.zeros((16,), F32)
        izeros16 = jnp.zeros((16,), jnp.int32)
        for p in range(passes):
            base = (wid * passes + p) * rows
            base_v = izeros16 + base

            def zero_vec(v, _):
                tile_v[pl.ds(v * 16, 16)] = zeros16
                return 0

            lax.fori_loop(0, rows * N // 16, zero_vec, 0)

            for c in range(nch):
                pltpu.sync_copy(dst_hbm.at[pl.ds(c * ch, ch)], dst_v)
                pltpu.sync_copy(src_hbm.at[pl.ds(c * ch, ch)], src_v)

                def edge_vec(j, _):
                    d = dst_v[pl.ds(j * 16, 16)]
                    s = src_v[pl.ds(j * 16, 16)]
                    rel = d - base_v
                    inr = (rel >= izeros16) & (rel < izeros16 + rows)
                    fi = jnp.where(inr, rel * N + s, izeros16)
                    cnt = jnp.sum(inr.astype(jnp.int32))

                    def one_hit():
                        plsc.addupdate_scatter(tile_v, [fi], ones, mask=inr)

                    def multi_hit():
                        for t in range(16):
                            plsc.addupdate_scatter(
                                tile_v, [fi], ones,
                                mask=inr & (lane == izeros16 + t),
                            )

                    def any_hit():
                        lax.cond(cnt > 1, multi_hit, one_hit)

                    lax.cond(cnt > 0, any_hit, lambda: None)
                    return 0

                lax.fori_loop(0, ch // 16, edge_vec, 0)

            pltpu.sync_copy(tile_v, out_hbm.at[pl.ds(base * N, rows * N)])

    return adj_kernel(dst, src).reshape(N, N)


def _deg_body(a_ref, dis_ref):
    deg = jnp.sum(a_ref[...], axis=1, keepdims=True) + 2.0
    dis_ref[...] = lax.rsqrt(deg)


def _norm_body(a_ref, dis_ref, dist_ref, an_ref, aa_ref):
    i = pl.program_id(0)
    r_i = lax.broadcasted_iota(jnp.int32, (BLK, N), 0) + i * BLK
    c_i = lax.broadcasted_iota(jnp.int32, (BLK, N), 1)
    eye = (r_i == c_i).astype(F32)
    a = a_ref[...]
    an_ref[...] = dis_ref[...] * (a + 2.0 * eye) * dist_ref[...]
    aa_ref[...] = a + eye


def _aug_body(aa_blk_ref, aa_ref, out_ref):
    i = pl.program_id(0)
    c = jnp.dot(aa_blk_ref[...], aa_ref[...], preferred_element_type=F32)
    r_i = lax.broadcasted_iota(jnp.int32, (BLK, N), 0) + i * BLK
    c_i = lax.broadcasted_iota(jnp.int32, (BLK, N), 1)
    out_ref[...] = jnp.where(r_i == c_i, 0.0, c)


def _ka_body(x_ref, an_ref, w0_ref, b0_ref, p_ref, w1_ref,
             h_ref, zp_ref, m_ref):
    z0 = jnp.dot(x_ref[...], w0_ref[...], preferred_element_type=F32)
    h = jnp.maximum(
        jnp.dot(an_ref[...], z0, preferred_element_type=F32) + b0_ref[...],
        0.0,
    )
    h_ref[...] = h
    p = p_ref[...]
    pn = p * lax.rsqrt(jnp.sum(p * p))
    score = jnp.dot(h, pn, preferred_element_type=F32)

    u = lax.bitcast_convert_type(score, jnp.int32)
    key = jnp.where(u >= 0, u, u ^ jnp.int32(0x7FFFFFFF))
    cnt0 = jnp.sum((key >= 0).astype(jnp.int32))
    t = jnp.where(cnt0 >= TOPK, jnp.int32(0), jnp.int32(-2147483648))
    for b in range(30, -1, -1):
        cand = t + jnp.int32(1 << b)
        cnt = jnp.sum((key >= cand).astype(jnp.int32))
        t = jnp.where(cnt >= TOPK, cand, t)
    cgt = jnp.sum((key > t).astype(jnp.int32))
    r = TOPK - cgt
    tie = key == t
    idx = lax.broadcasted_iota(jnp.int32, (N, 1), 0)
    tt = jnp.int32(-1)
    for b in range(10, -1, -1):
        cand = tt + jnp.int32(1 << b)
        g = jnp.sum((tie & (idx <= cand)).astype(jnp.int32))
        tt = jnp.where(g < r, cand, tt)
    sel = (key > t) | (tie & (idx <= tt + 1))
    m = sel.astype(F32)
    m_ref[...] = m
    s = m * jnp.tanh(score)
    zp_ref[...] = jnp.dot(h * s, w1_ref[...], preferred_element_type=F32)


def _kb_body(aug_ref, m_ref, zp_ref, h_ref, b1_ref, xn_ref):
    aug = aug_ref[...]
    m = m_ref[...]
    zp = zp_ref[...]
    degv = jnp.dot(aug, m, preferred_element_type=F32)
    w = m * lax.rsqrt(2.0 + degv)
    y = jnp.dot(aug, w * zp, preferred_element_type=F32)
    hp = jnp.maximum(w * y + 2.0 * (w * w) * zp + b1_ref[...], 0.0)
    xn_ref[...] = h_ref[...] + m * hp


def _kc_body(an_ref, xn_ref, wu_ref, bu_ref, o_ref):
    z2 = jnp.dot(xn_ref[...], wu_ref[...], preferred_element_type=F32)
    o_ref[...] = (
        jnp.dot(an_ref[...], z2, preferred_element_type=F32) + bu_ref[...]
    )


def _kc_bn_body(an_ref, xn_ref, wu_ref, bu_ref, g_ref, bb_ref, o_ref):
    z2 = jnp.dot(xn_ref[...], wu_ref[...], preferred_element_type=F32)
    o = jnp.dot(an_ref[...], z2, preferred_element_type=F32) + bu_ref[...]
    mu = jnp.mean(o, axis=0, keepdims=True)
    va = jnp.mean((o - mu) ** 2, axis=0, keepdims=True)
    o_ref[...] = g_ref[...] * (o - mu) / jnp.sqrt(va + EPS) + bb_ref[...]


def _unet_block(x, an, aug, pr, bn):
    cin = x.shape[1]
    hid = pr['W1'].shape[0]
    h, zp, m = pl.pallas_call(
        _ka_body,
        out_shape=[
            jax.ShapeDtypeStruct((N, hid), F32),
            jax.ShapeDtypeStruct((N, hid), F32),
            jax.ShapeDtypeStruct((N, 1), F32),
        ],
    )(x, an, pr['W0'], pr['b0'].reshape(1, hid), pr['p'].reshape(hid, 1),
      pr['W1'])
    xn = pl.pallas_call(
        _kb_body,
        out_shape=jax.ShapeDtypeStruct((N, hid), F32),
    )(aug, m, zp, h, pr['b1'].reshape(1, hid))
    cout = pr['Wu'].shape[1]
    if bn is None:
        out = pl.pallas_call(
            _kc_body,
            out_shape=jax.ShapeDtypeStruct((N, cout), F32),
        )(an, xn, pr['Wu'], pr['bu'].reshape(1, cout))
    else:
        out = pl.pallas_call(
            _kc_bn_body,
            out_shape=jax.ShapeDtypeStruct((N, cout), F32),
        )(an, xn, pr['Wu'], pr['bu'].reshape(1, cout),
          bn[0].reshape(1, cout), bn[1].reshape(1, cout))
    return out


def _forward_from_adj(a, x, params):
    dis = pl.pallas_call(
        _deg_body,
        grid=(N // BLK,),
        in_specs=[pl.BlockSpec((BLK, N), lambda i: (i, 0))],
        out_specs=pl.BlockSpec((BLK, 1), lambda i: (i, 0)),
        out_shape=jax.ShapeDtypeStruct((N, 1), F32),
    )(a)
    an, aa = pl.pallas_call(
        _norm_body,
        grid=(N // BLK,),
        in_specs=[
            pl.BlockSpec((BLK, N), lambda i: (i, 0)),
            pl.BlockSpec((BLK, 1), lambda i: (i, 0)),
            pl.BlockSpec((1, N), lambda i: (0, 0)),
        ],
        out_specs=[
            pl.BlockSpec((BLK, N), lambda i: (i, 0)),
            pl.BlockSpec((BLK, N), lambda i: (i, 0)),
        ],
        out_shape=[
            jax.ShapeDtypeStruct((N, N), F32),
            jax.ShapeDtypeStruct((N, N), F32),
        ],
    )(a, dis, dis.reshape(1, N))
    aug = pl.pallas_call(
        _aug_body,
        grid=(N // BLK,),
        in_specs=[
            pl.BlockSpec((BLK, N), lambda i: (i, 0)),
            pl.BlockSpec((N, N), lambda i: (0, 0)),
        ],
        out_specs=pl.BlockSpec((BLK, N), lambda i: (i, 0)),
        out_shape=jax.ShapeDtypeStruct((N, N), F32),
    )(aa, aa)

    depth = 3
    xs = []
    h = x
    for i in range(depth):
        h = _unet_block(
            h, an, aug, params['enc%d' % i],
            (params['bn%d_g' % i], params['bn%d_b' % i]),
        )
        xs.append(h)
    for i in range(depth - 1, -1, -1):
        h = _unet_block(
            jnp.concatenate([h, xs[i]], axis=1), an, aug,
            params['dec%d' % i], None,
        )
    return h


def kernel(x, params, edge_index):
    a = _build_adj(edge_index[1], edge_index[0])
    return _forward_from_adj(a, x, params)


# speedup vs baseline: 1.1182x; 1.1182x over previous
import functools

import jax
import jax.numpy as jnp
from jax import lax
from jax.experimental import pallas as pl
from jax.experimental.pallas import tpu as pltpu
from jax.experimental.pallas import tpu_sc as plsc

N = 2048
E = 32768
EPS = 1e-5
TOPK = (N + 1) // 2
BLK = 256
F32 = jnp.float32


def _build_adj(dst, src):
    info = plsc.get_sparse_core_info()
    nw = info.num_cores * info.num_subcores
    rows = 32
    passes = N // (nw * rows)
    ch = 16384
    nch = E // ch
    mesh = plsc.VectorSubcoreMesh(core_axis_name="c", subcore_axis_name="s")

    @functools.partial(
        pl.kernel,
        mesh=mesh,
        compiler_params=pltpu.CompilerParams(needs_layout_passes=False),
        out_type=jax.ShapeDtypeStruct((N * N,), F32),
        scratch_types=[
            pltpu.VMEM((rows * N,), F32),
            pltpu.VMEM((ch,), jnp.int32),
            pltpu.VMEM((ch,), jnp.int32),
        ],
    )
    def adj_kernel(dst_hbm, src_hbm, out_hbm, tile_v, dst_v, src_v):
        wid = lax.axis_index("s") * info.num_cores + lax.axis_index("c")
        lane = lax.iota(jnp.int32, 16)
        ones = jnp.ones((16,), F32)
        zeros16 = jnp.zeros((16,), F32)
        izeros16 = jnp.zeros((16,), jnp.int32)
        for p in range(passes):
            base = (wid * passes + p) * rows
            base_v = izeros16 + base

            def zero_vec(v, _):
                tile_v[pl.ds(v * 16, 16)] = zeros16
                return 0

            lax.fori_loop(0, rows * N // 16, zero_vec, 0)

            for c in range(nch):
                pltpu.sync_copy(dst_hbm.at[pl.ds(c * ch, ch)], dst_v)
                pltpu.sync_copy(src_hbm.at[pl.ds(c * ch, ch)], src_v)

                def edge_vec(j, _):
                    d = dst_v[pl.ds(j * 16, 16)]
                    s = src_v[pl.ds(j * 16, 16)]
                    rel = d - base_v
                    inr = (rel >= izeros16) & (rel < izeros16 + rows)
                    fi = jnp.where(inr, rel * N + s, izeros16)
                    for t in range(16):
                        plsc.addupdate_scatter(
                            tile_v, [fi], ones,
                            mask=inr & (lane == izeros16 + t),
                        )
                    return 0

                lax.fori_loop(0, ch // 16, edge_vec, 0)

            pltpu.sync_copy(tile_v, out_hbm.at[pl.ds(base * N, rows * N)])

    return adj_kernel(dst, src).reshape(N, N)


def _deg_body(a_ref, dis_ref):
    deg = jnp.sum(a_ref[...], axis=1, keepdims=True) + 2.0
    dis_ref[...] = lax.rsqrt(deg)


def _norm_body(a_ref, dis_ref, dist_ref, an_ref, aa_ref):
    i = pl.program_id(0)
    r_i = lax.broadcasted_iota(jnp.int32, (BLK, N), 0) + i * BLK
    c_i = lax.broadcasted_iota(jnp.int32, (BLK, N), 1)
    eye = (r_i == c_i).astype(F32)
    a = a_ref[...]
    an_ref[...] = dis_ref[...] * (a + 2.0 * eye) * dist_ref[...]
    aa_ref[...] = (a + eye).astype(jnp.bfloat16)


def _aug_body(aa_blk_ref, aa_ref, out_ref):
    i = pl.program_id(0)
    c = jnp.dot(aa_blk_ref[...], aa_ref[...], preferred_element_type=F32)
    r_i = lax.broadcasted_iota(jnp.int32, (BLK, N), 0) + i * BLK
    c_i = lax.broadcasted_iota(jnp.int32, (BLK, N), 1)
    out_ref[...] = jnp.where(r_i == c_i, 0.0, c).astype(jnp.bfloat16)


def _ka_body(x_ref, an_ref, w0_ref, b0_ref, p_ref, w1_ref,
             h_ref, zp_ref, m_ref):
    z0 = jnp.dot(x_ref[...], w0_ref[...], preferred_element_type=F32)
    h = jnp.maximum(
        jnp.dot(an_ref[...], z0, preferred_element_type=F32) + b0_ref[...],
        0.0,
    )
    h_ref[...] = h
    p = p_ref[...]
    pn = p * lax.rsqrt(jnp.sum(p * p))
    score = jnp.dot(h, pn, preferred_element_type=F32)

    u = lax.bitcast_convert_type(score, jnp.int32)
    key = jnp.where(u >= 0, u, u ^ jnp.int32(0x7FFFFFFF))
    cnt0 = jnp.sum((key >= 0).astype(jnp.int32))
    t = jnp.where(cnt0 >= TOPK, jnp.int32(0), jnp.int32(-2147483648))
    for b in range(30, -1, -1):
        cand = t + jnp.int32(1 << b)
        cnt = jnp.sum((key >= cand).astype(jnp.int32))
        t = jnp.where(cnt >= TOPK, cand, t)
    cgt = jnp.sum((key > t).astype(jnp.int32))
    r = TOPK - cgt
    tie = key == t
    idx = lax.broadcasted_iota(jnp.int32, (N, 1), 0)
    tt = jnp.int32(-1)
    for b in range(10, -1, -1):
        cand = tt + jnp.int32(1 << b)
        g = jnp.sum((tie & (idx <= cand)).astype(jnp.int32))
        tt = jnp.where(g < r, cand, tt)
    sel = (key > t) | (tie & (idx <= tt + 1))
    m = sel.astype(F32)
    m_ref[...] = m
    s = m * jnp.tanh(score)
    zp_ref[...] = jnp.dot(h * s, w1_ref[...], preferred_element_type=F32)


def _kb_body(aug_ref, m_ref, zp_ref, h_ref, b1_ref, xn_ref):
    aug = aug_ref[...]
    m = m_ref[...]
    zp = zp_ref[...]
    degv = jnp.dot(aug, m.astype(jnp.bfloat16), preferred_element_type=F32)
    w = m * lax.rsqrt(2.0 + degv)
    y = jnp.dot(aug, (w * zp).astype(jnp.bfloat16),
                preferred_element_type=F32)
    hp = jnp.maximum(w * y + 2.0 * (w * w) * zp + b1_ref[...], 0.0)
    xn_ref[...] = h_ref[...] + m * hp


def _kc_body(an_ref, xn_ref, wu_ref, bu_ref, o_ref):
    z2 = jnp.dot(xn_ref[...], wu_ref[...], preferred_element_type=F32)
    o_ref[...] = (
        jnp.dot(an_ref[...], z2, preferred_element_type=F32) + bu_ref[...]
    )


def _kc_bn_body(an_ref, xn_ref, wu_ref, bu_ref, g_ref, bb_ref, o_ref):
    z2 = jnp.dot(xn_ref[...], wu_ref[...], preferred_element_type=F32)
    o = jnp.dot(an_ref[...], z2, preferred_element_type=F32) + bu_ref[...]
    mu = jnp.mean(o, axis=0, keepdims=True)
    va = jnp.mean((o - mu) ** 2, axis=0, keepdims=True)
    o_ref[...] = g_ref[...] * (o - mu) / jnp.sqrt(va + EPS) + bb_ref[...]


def _unet_block(x, an, aug, pr, bn):
    cin = x.shape[1]
    hid = pr['W1'].shape[0]
    h, zp, m = pl.pallas_call(
        _ka_body,
        out_shape=[
            jax.ShapeDtypeStruct((N, hid), F32),
            jax.ShapeDtypeStruct((N, hid), F32),
            jax.ShapeDtypeStruct((N, 1), F32),
        ],
    )(x, an, pr['W0'], pr['b0'].reshape(1, hid), pr['p'].reshape(hid, 1),
      pr['W1'])
    xn = pl.pallas_call(
        _kb_body,
        out_shape=jax.ShapeDtypeStruct((N, hid), F32),
    )(aug, m, zp, h, pr['b1'].reshape(1, hid))
    cout = pr['Wu'].shape[1]
    if bn is None:
        out = pl.pallas_call(
            _kc_body,
            out_shape=jax.ShapeDtypeStruct((N, cout), F32),
        )(an, xn, pr['Wu'], pr['bu'].reshape(1, cout))
    else:
        out = pl.pallas_call(
            _kc_bn_body,
            out_shape=jax.ShapeDtypeStruct((N, cout), F32),
        )(an, xn, pr['Wu'], pr['bu'].reshape(1, cout),
          bn[0].reshape(1, cout), bn[1].reshape(1, cout))
    return out


def _forward_from_adj(a, x, params):
    dis = pl.pallas_call(
        _deg_body,
        grid=(N // BLK,),
        in_specs=[pl.BlockSpec((BLK, N), lambda i: (i, 0))],
        out_specs=pl.BlockSpec((BLK, 1), lambda i: (i, 0)),
        out_shape=jax.ShapeDtypeStruct((N, 1), F32),
    )(a)
    an, aa = pl.pallas_call(
        _norm_body,
        grid=(N // BLK,),
        in_specs=[
            pl.BlockSpec((BLK, N), lambda i: (i, 0)),
            pl.BlockSpec((BLK, 1), lambda i: (i, 0)),
            pl.BlockSpec((1, N), lambda i: (0, 0)),
        ],
        out_specs=[
            pl.BlockSpec((BLK, N), lambda i: (i, 0)),
            pl.BlockSpec((BLK, N), lambda i: (i, 0)),
        ],
        out_shape=[
            jax.ShapeDtypeStruct((N, N), F32),
            jax.ShapeDtypeStruct((N, N), jnp.bfloat16),
        ],
    )(a, dis, dis.reshape(1, N))
    aug = pl.pallas_call(
        _aug_body,
        grid=(N // BLK,),
        in_specs=[
            pl.BlockSpec((BLK, N), lambda i: (i, 0)),
            pl.BlockSpec((N, N), lambda i: (0, 0)),
        ],
        out_specs=pl.BlockSpec((BLK, N), lambda i: (i, 0)),
        out_shape=jax.ShapeDtypeStruct((N, N), jnp.bfloat16),
    )(aa, aa)

    depth = 3
    xs = []
    h = x
    for i in range(depth):
        h = _unet_block(
            h, an, aug, params['enc%d' % i],
            (params['bn%d_g' % i], params['bn%d_b' % i]),
        )
        xs.append(h)
    for i in range(depth - 1, -1, -1):
        h = _unet_block(
            jnp.concatenate([h, xs[i]], axis=1), an, aug,
            params['dec%d' % i], None,
        )
    return h


def kernel(x, params, edge_index):
    a = _build_adj(edge_index[1], edge_index[0])
    return _forward_from_adj(a, x, params)


# speedup vs baseline: 1.2318x; 1.1016x over previous
import functools

import jax
import jax.numpy as jnp
from jax import lax
from jax.experimental import pallas as pl
from jax.experimental.pallas import tpu as pltpu
from jax.experimental.pallas import tpu_sc as plsc

N = 2048
E = 32768
EPS = 1e-5
TOPK = (N + 1) // 2
BLK = 256
F32 = jnp.float32


def _build_adj(dst, src):
    info = plsc.get_sparse_core_info()
    nw = info.num_cores * info.num_subcores
    rows = 32
    passes = N // (nw * rows)
    ch = 16384
    nch = E // ch
    mesh = plsc.VectorSubcoreMesh(core_axis_name="c", subcore_axis_name="s")

    @functools.partial(
        pl.kernel,
        mesh=mesh,
        compiler_params=pltpu.CompilerParams(needs_layout_passes=False),
        out_type=jax.ShapeDtypeStruct((N * N,), F32),
        scratch_types=[
            pltpu.VMEM((rows * N,), F32),
            pltpu.VMEM((ch,), jnp.int32),
            pltpu.VMEM((ch,), jnp.int32),
        ],
    )
    def adj_kernel(dst_hbm, src_hbm, out_hbm, tile_v, dst_v, src_v):
        wid = lax.axis_index("s") * info.num_cores + lax.axis_index("c")
        lane = lax.iota(jnp.int32, 16)
        ones = jnp.ones((16,), F32)
        zeros16 = jnp.zeros((16,), F32)
        izeros16 = jnp.zeros((16,), jnp.int32)
        for p in range(passes):
            base = (wid * passes + p) * rows
            base_v = izeros16 + base

            def zero_vec(v, _):
                tile_v[pl.ds(v * 16, 16)] = zeros16
                return 0

            lax.fori_loop(0, rows * N // 16, zero_vec, 0)

            for c in range(nch):
                pltpu.sync_copy(dst_hbm.at[pl.ds(c * ch, ch)], dst_v)
                pltpu.sync_copy(src_hbm.at[pl.ds(c * ch, ch)], src_v)

                def edge_vec(j, _):
                    d = dst_v[pl.ds(j * 16, 16)]
                    s = src_v[pl.ds(j * 16, 16)]
                    rel = d - base_v
                    inr = (rel >= izeros16) & (rel < izeros16 + rows)
                    fi = jnp.where(inr, rel * N + s, izeros16)
                    for t in range(16):
                        plsc.addupdate_scatter(
                            tile_v, [fi], ones,
                            mask=inr & (lane == izeros16 + t),
                        )
                    return 0

                lax.fori_loop(0, ch // 16, edge_vec, 0)

            pltpu.sync_copy(tile_v, out_hbm.at[pl.ds(base * N, rows * N)])

    return adj_kernel(dst, src).reshape(N, N)


def _deg_body(a_ref, dis_ref):
    deg = jnp.sum(a_ref[...], axis=1, keepdims=True) + 2.0
    dis_ref[...] = lax.rsqrt(deg)


def _norm_body(a_ref, dis_ref, dist_ref, an_ref, aa_ref):
    i = pl.program_id(0)
    r_i = lax.broadcasted_iota(jnp.int32, (BLK, N), 0) + i * BLK
    c_i = lax.broadcasted_iota(jnp.int32, (BLK, N), 1)
    eye = (r_i == c_i).astype(F32)
    a = a_ref[...]
    an_ref[...] = dis_ref[...] * (a + 2.0 * eye) * dist_ref[...]
    aa_ref[...] = (a + eye).astype(jnp.bfloat16)


def _aug_body(aa_blk_ref, aa_ref, out_ref):
    i = pl.program_id(0)
    c = jnp.dot(aa_blk_ref[...], aa_ref[...], preferred_element_type=F32)
    r_i = lax.broadcasted_iota(jnp.int32, (BLK, N), 0) + i * BLK
    c_i = lax.broadcasted_iota(jnp.int32, (BLK, N), 1)
    out_ref[...] = jnp.where(r_i == c_i, 0.0, c).astype(jnp.bfloat16)


def _block_math(x_ref, an_ref, aug_ref, w0_ref, b0_ref, p_ref, w1_ref,
                b1_ref, wu_ref, bu_ref):
    z0 = jnp.dot(x_ref[...], w0_ref[...], preferred_element_type=F32)
    h = jnp.maximum(
        jnp.dot(an_ref[...], z0, preferred_element_type=F32) + b0_ref[...],
        0.0,
    )
    p = p_ref[...]
    pn = p * lax.rsqrt(jnp.sum(p * p))
    score = jnp.dot(h, pn, preferred_element_type=F32)

    u = lax.bitcast_convert_type(score, jnp.int32)
    key = jnp.where(u >= 0, u, u ^ jnp.int32(0x7FFFFFFF))
    cnt0 = jnp.sum((key >= 0).astype(jnp.int32))
    t = jnp.where(cnt0 >= TOPK, jnp.int32(0), jnp.int32(-2147483648))
    for b in range(30, -1, -1):
        cand = t + jnp.int32(1 << b)
        cnt = jnp.sum((key >= cand).astype(jnp.int32))
        t = jnp.where(cnt >= TOPK, cand, t)
    cgt = jnp.sum((key > t).astype(jnp.int32))
    r = TOPK - cgt
    tie = key == t
    idx = lax.broadcasted_iota(jnp.int32, (N, 1), 0)
    tt = jnp.int32(-1)
    for b in range(10, -1, -1):
        cand = tt + jnp.int32(1 << b)
        g = jnp.sum((tie & (idx <= cand)).astype(jnp.int32))
        tt = jnp.where(g < r, cand, tt)
    sel = (key > t) | (tie & (idx <= tt + 1))
    m = sel.astype(F32)
    s = m * jnp.tanh(score)
    zp = jnp.dot(h * s, w1_ref[...], preferred_element_type=F32)

    aug = aug_ref[...]
    degv = jnp.dot(aug, m.astype(jnp.bfloat16), preferred_element_type=F32)
    w = m * lax.rsqrt(2.0 + degv)
    y = jnp.dot(aug, (w * zp).astype(jnp.bfloat16),
                preferred_element_type=F32)
    hp = jnp.maximum(w * y + 2.0 * (w * w) * zp + b1_ref[...], 0.0)
    xn = h + m * hp

    z2 = jnp.dot(xn, wu_ref[...], preferred_element_type=F32)
    return (
        jnp.dot(an_ref[...], z2, preferred_element_type=F32) + bu_ref[...]
    )


def _block_body(*refs):
    o_ref = refs[-1]
    o_ref[...] = _block_math(*refs[:-1])


def _block_bn_body(*refs):
    o_ref = refs[-1]
    g_ref, bb_ref = refs[-3], refs[-2]
    o = _block_math(*refs[:-3])
    mu = jnp.mean(o, axis=0, keepdims=True)
    va = jnp.mean((o - mu) ** 2, axis=0, keepdims=True)
    o_ref[...] = g_ref[...] * (o - mu) / jnp.sqrt(va + EPS) + bb_ref[...]


def _unet_block(x, an, aug, pr, bn):
    hid = pr['W1'].shape[0]
    cout = pr['Wu'].shape[1]
    args = (x, an, aug, pr['W0'], pr['b0'].reshape(1, hid),
            pr['p'].reshape(hid, 1), pr['W1'], pr['b1'].reshape(1, hid),
            pr['Wu'], pr['bu'].reshape(1, cout))
    if bn is None:
        return pl.pallas_call(
            _block_body,
            out_shape=jax.ShapeDtypeStruct((N, cout), F32),
        )(*args)
    return pl.pallas_call(
        _block_bn_body,
        out_shape=jax.ShapeDtypeStruct((N, cout), F32),
    )(*args, bn[0].reshape(1, cout), bn[1].reshape(1, cout))


def _forward_from_adj(a, x, params):
    dis = pl.pallas_call(
        _deg_body,
        grid=(N // BLK,),
        in_specs=[pl.BlockSpec((BLK, N), lambda i: (i, 0))],
        out_specs=pl.BlockSpec((BLK, 1), lambda i: (i, 0)),
        out_shape=jax.ShapeDtypeStruct((N, 1), F32),
    )(a)
    an, aa = pl.pallas_call(
        _norm_body,
        grid=(N // BLK,),
        in_specs=[
            pl.BlockSpec((BLK, N), lambda i: (i, 0)),
            pl.BlockSpec((BLK, 1), lambda i: (i, 0)),
            pl.BlockSpec((1, N), lambda i: (0, 0)),
        ],
        out_specs=[
            pl.BlockSpec((BLK, N), lambda i: (i, 0)),
            pl.BlockSpec((BLK, N), lambda i: (i, 0)),
        ],
        out_shape=[
            jax.ShapeDtypeStruct((N, N), F32),
            jax.ShapeDtypeStruct((N, N), jnp.bfloat16),
        ],
    )(a, dis, dis.reshape(1, N))
    aug = pl.pallas_call(
        _aug_body,
        grid=(N // BLK,),
        in_specs=[
            pl.BlockSpec((BLK, N), lambda i: (i, 0)),
            pl.BlockSpec((N, N), lambda i: (0, 0)),
        ],
        out_specs=pl.BlockSpec((BLK, N), lambda i: (i, 0)),
        out_shape=jax.ShapeDtypeStruct((N, N), jnp.bfloat16),
    )(aa, aa)

    depth = 3
    xs = []
    h = x
    for i in range(depth):
        h = _unet_block(
            h, an, aug, params['enc%d' % i],
            (params['bn%d_g' % i], params['bn%d_b' % i]),
        )
        xs.append(h)
    for i in range(depth - 1, -1, -1):
        h = _unet_block(
            jnp.concatenate([h, xs[i]], axis=1), an, aug,
            params['dec%d' % i], None,
        )
    return h


def kernel(x, params, edge_index):
    a = _build_adj(edge_index[1], edge_index[0])
    return _forward_from_adj(a, x, params)


# speedup vs baseline: 1.2549x; 1.0188x over previous
import functools

import jax
import jax.numpy as jnp
from jax import lax
from jax.experimental import pallas as pl
from jax.experimental.pallas import tpu as pltpu
from jax.experimental.pallas import tpu_sc as plsc

N = 2048
E = 32768
EPS = 1e-5
TOPK = (N + 1) // 2
BLK = 256
F32 = jnp.float32


def _build_adj(dst, src):
    info = plsc.get_sparse_core_info()
    nw = info.num_cores * info.num_subcores
    rows = 32
    passes = N // (nw * rows)
    ch = 16384
    nch = E // ch
    mesh = plsc.VectorSubcoreMesh(core_axis_name="c", subcore_axis_name="s")

    @functools.partial(
        pl.kernel,
        mesh=mesh,
        compiler_params=pltpu.CompilerParams(needs_layout_passes=False),
        out_type=jax.ShapeDtypeStruct((N * N,), F32),
        scratch_types=[
            pltpu.VMEM((rows * N,), F32),
            pltpu.VMEM((ch,), jnp.int32),
            pltpu.VMEM((ch,), jnp.int32),
        ],
    )
    def adj_kernel(dst_hbm, src_hbm, out_hbm, tile_v, dst_v, src_v):
        wid = lax.axis_index("s") * info.num_cores + lax.axis_index("c")
        lane = lax.iota(jnp.int32, 16)
        ones = jnp.ones((16,), F32)
        zeros16 = jnp.zeros((16,), F32)
        izeros16 = jnp.zeros((16,), jnp.int32)
        for p in range(passes):
            base = (wid * passes + p) * rows
            base_v = izeros16 + base

            def zero_vec(v, _):
                tile_v[pl.ds(v * 16, 16)] = zeros16
                return 0

            lax.fori_loop(0, rows * N // 16, zero_vec, 0)

            for c in range(nch):
                pltpu.sync_copy(dst_hbm.at[pl.ds(c * ch, ch)], dst_v)
                pltpu.sync_copy(src_hbm.at[pl.ds(c * ch, ch)], src_v)

                def edge_vec(j, _):
                    d = dst_v[pl.ds(j * 16, 16)]
                    s = src_v[pl.ds(j * 16, 16)]
                    rel = d - base_v
                    inr = (rel >= izeros16) & (rel < izeros16 + rows)
                    fi = jnp.where(inr, rel * N + s, izeros16)
                    cnts, last = plsc.scan_count(fi, mask=inr)
                    plsc.addupdate_scatter(
                        tile_v, [fi], cnts.astype(F32), mask=last & inr
                    )
                    return 0

                lax.fori_loop(0, ch // 16, edge_vec, 0)

            pltpu.sync_copy(tile_v, out_hbm.at[pl.ds(base * N, rows * N)])

    return adj_kernel(dst, src).reshape(N, N)


def _deg_body(a_ref, dis_ref):
    deg = jnp.sum(a_ref[...], axis=1, keepdims=True) + 2.0
    dis_ref[...] = lax.rsqrt(deg)


def _norm_body(a_ref, dis_ref, dist_ref, an_ref, aa_ref):
    i = pl.program_id(0)
    r_i = lax.broadcasted_iota(jnp.int32, (BLK, N), 0) + i * BLK
    c_i = lax.broadcasted_iota(jnp.int32, (BLK, N), 1)
    eye = (r_i == c_i).astype(F32)
    a = a_ref[...]
    an_ref[...] = dis_ref[...] * (a + 2.0 * eye) * dist_ref[...]
    aa_ref[...] = (a + eye).astype(jnp.bfloat16)


def _aug_body(aa_blk_ref, aa_ref, out_ref):
    i = pl.program_id(0)
    c = jnp.dot(aa_blk_ref[...], aa_ref[...], preferred_element_type=F32)
    r_i = lax.broadcasted_iota(jnp.int32, (BLK, N), 0) + i * BLK
    c_i = lax.broadcasted_iota(jnp.int32, (BLK, N), 1)
    out_ref[...] = jnp.where(r_i == c_i, 0.0, c).astype(jnp.bfloat16)


def _block_math(x_ref, an_ref, aug_ref, w0_ref, b0_ref, p_ref, w1_ref,
                b1_ref, wu_ref, bu_ref):
    z0 = jnp.dot(x_ref[...], w0_ref[...], preferred_element_type=F32)
    h = jnp.maximum(
        jnp.dot(an_ref[...], z0, preferred_element_type=F32) + b0_ref[...],
        0.0,
    )
    p = p_ref[...]
    pn = p * lax.rsqrt(jnp.sum(p * p))
    score = jnp.dot(h, pn, preferred_element_type=F32)

    u = lax.bitcast_convert_type(score, jnp.int32)
    key = jnp.where(u >= 0, u, u ^ jnp.int32(0x7FFFFFFF))
    cnt0 = jnp.sum((key >= 0).astype(jnp.int32))
    t = jnp.where(cnt0 >= TOPK, jnp.int32(0), jnp.int32(-2147483648))
    for b in range(30, -1, -1):
        cand = t + jnp.int32(1 << b)
        cnt = jnp.sum((key >= cand).astype(jnp.int32))
        t = jnp.where(cnt >= TOPK, cand, t)
    cgt = jnp.sum((key > t).astype(jnp.int32))
    r = TOPK - cgt
    tie = key == t
    idx = lax.broadcasted_iota(jnp.int32, (N, 1), 0)
    tt = jnp.int32(-1)
    for b in range(10, -1, -1):
        cand = tt + jnp.int32(1 << b)
        g = jnp.sum((tie & (idx <= cand)).astype(jnp.int32))
        tt = jnp.where(g < r, cand, tt)
    sel = (key > t) | (tie & (idx <= tt + 1))
    m = sel.astype(F32)
    s = m * jnp.tanh(score)
    zp = jnp.dot(h * s, w1_ref[...], preferred_element_type=F32)

    aug = aug_ref[...]
    degv = jnp.dot(aug, m.astype(jnp.bfloat16), preferred_element_type=F32)
    w = m * lax.rsqrt(2.0 + degv)
    y = jnp.dot(aug, (w * zp).astype(jnp.bfloat16),
                preferred_element_type=F32)
    hp = jnp.maximum(w * y + 2.0 * (w * w) * zp + b1_ref[...], 0.0)
    xn = h + m * hp

    z2 = jnp.dot(xn, wu_ref[...], preferred_element_type=F32)
    return (
        jnp.dot(an_ref[...], z2, preferred_element_type=F32) + bu_ref[...]
    )


def _block_body(*refs):
    o_ref = refs[-1]
    o_ref[...] = _block_math(*refs[:-1])


def _block_bn_body(*refs):
    o_ref = refs[-1]
    g_ref, bb_ref = refs[-3], refs[-2]
    o = _block_math(*refs[:-3])
    mu = jnp.mean(o, axis=0, keepdims=True)
    va = jnp.mean((o - mu) ** 2, axis=0, keepdims=True)
    o_ref[...] = g_ref[...] * (o - mu) / jnp.sqrt(va + EPS) + bb_ref[...]


def _unet_block(x, an, aug, pr, bn):
    hid = pr['W1'].shape[0]
    cout = pr['Wu'].shape[1]
    args = (x, an, aug, pr['W0'], pr['b0'].reshape(1, hid),
            pr['p'].reshape(hid, 1), pr['W1'], pr['b1'].reshape(1, hid),
            pr['Wu'], pr['bu'].reshape(1, cout))
    if bn is None:
        return pl.pallas_call(
            _block_body,
            out_shape=jax.ShapeDtypeStruct((N, cout), F32),
        )(*args)
    return pl.pallas_call(
        _block_bn_body,
        out_shape=jax.ShapeDtypeStruct((N, cout), F32),
    )(*args, bn[0].reshape(1, cout), bn[1].reshape(1, cout))


def _forward_from_adj(a, x, params):
    dis = pl.pallas_call(
        _deg_body,
        grid=(N // BLK,),
        in_specs=[pl.BlockSpec((BLK, N), lambda i: (i, 0))],
        out_specs=pl.BlockSpec((BLK, 1), lambda i: (i, 0)),
        out_shape=jax.ShapeDtypeStruct((N, 1), F32),
    )(a)
    an, aa = pl.pallas_call(
        _norm_body,
        grid=(N // BLK,),
        in_specs=[
            pl.BlockSpec((BLK, N), lambda i: (i, 0)),
            pl.BlockSpec((BLK, 1), lambda i: (i, 0)),
            pl.BlockSpec((1, N), lambda i: (0, 0)),
        ],
        out_specs=[
            pl.BlockSpec((BLK, N), lambda i: (i, 0)),
            pl.BlockSpec((BLK, N), lambda i: (i, 0)),
        ],
        out_shape=[
            jax.ShapeDtypeStruct((N, N), F32),
            jax.ShapeDtypeStruct((N, N), jnp.bfloat16),
        ],
    )(a, dis, dis.reshape(1, N))
    aug = pl.pallas_call(
        _aug_body,
        grid=(N // BLK,),
        in_specs=[
            pl.BlockSpec((BLK, N), lambda i: (i, 0)),
            pl.BlockSpec((N, N), lambda i: (0, 0)),
        ],
        out_specs=pl.BlockSpec((BLK, N), lambda i: (i, 0)),
        out_shape=jax.ShapeDtypeStruct((N, N), jnp.bfloat16),
    )(aa, aa)

    depth = 3
    xs = []
    h = x
    for i in range(depth):
        h = _unet_block(
            h, an, aug, params['enc%d' % i],
            (params['bn%d_g' % i], params['bn%d_b' % i]),
        )
        xs.append(h)
    for i in range(depth - 1, -1, -1):
        h = _unet_block(
            jnp.concatenate([h, xs[i]], axis=1), an, aug,
            params['dec%d' % i], None,
        )
    return h


def kernel(x, params, edge_index):
    a = _build_adj(edge_index[1], edge_index[0])
    return _forward_from_adj(a, x, params)
